# Initial kernel scaffold; baseline (speedup 1.0000x reference)
import jax
import jax.numpy as jnp
from jax.experimental import pallas as pl


def kernel(rnn_output, indices, targets, W, b):
    raise NotImplementedError("write your pallas kernel here")


# speedup vs baseline: 5.7759x; 5.7759x over previous
import functools

import jax
import jax.numpy as jnp
from jax import lax
from jax.experimental import pallas as pl
from jax.experimental.pallas import tpu as pltpu
from jax.experimental.pallas import tpu_sc as plsc

_SIZE = 1024
_ROWS = 32768
_N = 16384
_NC, _NS = 2, 16
_NW = _NC * _NS
_IDX_MINOR = 128
_IDX_ROWS = _N // _IDX_MINOR
_ROWS_PER_W = _IDX_ROWS // _NW

_MV_BLOCK = 2048


def _matvec_body(x_ref, w_ref, b_ref, o_ref):
    o_ref[...] = jnp.sum(x_ref[...] * w_ref[...], axis=1) + b_ref[0]


def _matvec(flat, W, b):
    return pl.pallas_call(
        _matvec_body,
        grid=(_ROWS // _MV_BLOCK,),
        in_specs=[
            pl.BlockSpec((_MV_BLOCK, _SIZE), lambda i: (i, 0)),
            pl.BlockSpec((1, _SIZE), lambda i: (0, 0)),
            pl.BlockSpec(memory_space=pltpu.SMEM),
        ],
        out_specs=pl.BlockSpec((_MV_BLOCK,), lambda i: (i,)),
        out_shape=jax.ShapeDtypeStruct((_ROWS,), jnp.float32),
    )(flat, W, b)


def _gather_body(p_hbm, idx_hbm, out_hbm, idx_v, vals_v, sem):
    wid = lax.axis_index("s") * _NC + lax.axis_index("c")
    base = wid * _ROWS_PER_W
    pltpu.sync_copy(idx_hbm.at[pl.ds(base, _ROWS_PER_W)], idx_v)
    copies = [
        pltpu.async_copy(p_hbm.at[idx_v.at[j]], vals_v.at[j], sem)
        for j in range(_ROWS_PER_W)
    ]
    for c in copies:
        c.wait()
    pltpu.sync_copy(vals_v, out_hbm.at[pl.ds(base, _ROWS_PER_W)])


def _sc_gather(p, idx):
    call = pl.kernel(
        _gather_body,
        out_type=jax.ShapeDtypeStruct((_IDX_ROWS, _IDX_MINOR), jnp.float32),
        mesh=plsc.VectorSubcoreMesh(core_axis_name="c", subcore_axis_name="s"),
        scratch_types=[
            pltpu.VMEM((_ROWS_PER_W, _IDX_MINOR), jnp.int32),
            pltpu.VMEM((_ROWS_PER_W, _IDX_MINOR), jnp.float32),
            pltpu.SemaphoreType.DMA,
        ],
    )
    return call(p, idx)


def _loss_body(f_ref, t_ref, o_ref):
    f = f_ref[...]
    t = t_ref[...]
    val = jnp.sum(jnp.maximum(f, 0.0) - f * t + jnp.log1p(jnp.exp(-jnp.abs(f))))
    o_ref[...] = val.reshape(1, 1)


def _loss(final2d, targets2d):
    return pl.pallas_call(
        _loss_body,
        out_shape=jax.ShapeDtypeStruct((1, 1), jnp.float32),
    )(final2d, targets2d)


def kernel(rnn_output, indices, targets, W, b):
    flat = rnn_output.reshape(_ROWS, _SIZE)
    idx = indices.astype(jnp.int32).reshape(_IDX_ROWS, _IDX_MINOR)
    p = _matvec(flat, W, b)
    final2d = _sc_gather(p, idx)
    loss = _loss(final2d, targets.reshape(_IDX_ROWS, _IDX_MINOR))
    return final2d.reshape(_N), loss.reshape(())
